# baseline (speedup 1.0000x reference)
import functools
import math

import jax
import jax.numpy as jnp
from jax import lax
from jax.experimental import pallas as pl
from jax.experimental.pallas import tpu as pltpu
from jax.experimental.pallas import tpu_sc as plsc

D = 2048
E = 8
K = 2
T = 2048
CAP = 640
S = E * CAP
NW = 32
SPW = S // NW
CHUNK = 16
NEG_INF = float("-inf")


def _m1_body(x_ref, wg_ref, e0_ref, e1_ref, p0_ref, p1_ref):
    xb = x_ref[...]
    wg = wg_ref[...]
    logits = lax.dot_general(xb, wg, (((1,), (1,)), ((), ())),
                             preferred_element_type=jnp.float32)
    cols = lax.broadcasted_iota(jnp.int32, logits.shape, 1)
    l0 = jnp.max(logits, axis=1, keepdims=True)
    e0 = jnp.min(jnp.where(logits == l0, cols, E), axis=1)
    rest = jnp.where(cols == e0[:, None], NEG_INF, logits)
    l1 = jnp.max(rest, axis=1, keepdims=True)
    e1 = jnp.min(jnp.where(rest == l1, cols, E), axis=1)
    z = jnp.exp(l1[:, 0] - l0[:, 0])
    p0 = 1.0 / (1.0 + z)
    p1 = z / (1.0 + z)
    n = xb.shape[0]
    e0_ref[...] = e0.reshape(1, n)
    e1_ref[...] = e1.reshape(1, n)
    p0_ref[...] = p0.reshape(1, n)
    p1_ref[...] = p1.reshape(1, n)


def _m1_call(xf, wg):
    blk = 256
    grid = T // blk
    out = pl.pallas_call(
        _m1_body,
        grid=(grid,),
        in_specs=[
            pl.BlockSpec((blk, D), lambda i: (i, 0)),
            pl.BlockSpec((E, D), lambda i: (0, 0)),
        ],
        out_specs=[
            pl.BlockSpec((1, blk), lambda i: (0, i)),
            pl.BlockSpec((1, blk), lambda i: (0, i)),
            pl.BlockSpec((1, blk), lambda i: (0, i)),
            pl.BlockSpec((1, blk), lambda i: (0, i)),
        ],
        out_shape=[
            jax.ShapeDtypeStruct((1, T), jnp.int32),
            jax.ShapeDtypeStruct((1, T), jnp.int32),
            jax.ShapeDtypeStruct((1, T), jnp.float32),
            jax.ShapeDtypeStruct((1, T), jnp.float32),
        ],
    )(xf, wg)
    return out


def _m2_body(e0_ref, e1_ref, p0_ref, p1_ref, s0_ref, s1_ref):
    e0 = e0_ref[0, :]
    e1 = e1_ref[0, :]
    cols8 = lax.broadcasted_iota(jnp.int32, (T, E), 1)
    oh0 = (cols8 == e0[:, None]).astype(jnp.float32)
    oh1 = (cols8 == e1[:, None]).astype(jnp.float32)

    blk = 256
    ii = lax.broadcasted_iota(jnp.int32, (blk, blk), 0)
    jj = lax.broadcasted_iota(jnp.int32, (blk, blk), 1)
    tri = (jj < ii).astype(jnp.float32)

    def cum(oh, run):
        outs = []
        for b in range(T // blk):
            piece = oh[b * blk:(b + 1) * blk, :]
            c = lax.dot_general(tri, piece, (((1,), (0,)), ((), ())),
                                preferred_element_type=jnp.float32) + run
            outs.append(c)
            run = run + jnp.sum(piece, axis=0, keepdims=True)
        return jnp.concatenate(outs, axis=0), run

    cum0, c0tot = cum(oh0, jnp.zeros((1, E), jnp.float32))
    cum1, _ = cum(oh1, c0tot)
    r0 = jnp.sum(cum0 * oh0, axis=1).astype(jnp.int32)
    r1 = jnp.sum(cum1 * oh1, axis=1).astype(jnp.int32)
    p0 = p0_ref[0, :]
    p1 = p1_ref[0, :]
    keep0 = (r0 < CAP) & (p0 != 0.0)
    keep1 = (r1 < CAP) & (p1 != 0.0)
    s0 = jnp.where(keep0, e0 * CAP + r0, -1)
    s1 = jnp.where(keep1, e1 * CAP + r1, -1)
    s0_ref[...] = s0.reshape(1, T)
    s1_ref[...] = s1.reshape(1, T)


def _m2_call(e0, e1, p0, p1):
    return pl.pallas_call(
        _m2_body,
        out_shape=[
            jax.ShapeDtypeStruct((1, T), jnp.int32),
            jax.ShapeDtypeStruct((1, T), jnp.int32),
        ],
    )(e0, e1, p0, p1)


def _dw_body(s0_ref, s1_ref, p0_ref, p1_ref, w_ref, m_ref):
    i = pl.program_id(0)
    n = w_ref.shape[0]
    sl = pl.ds(i * n, n)
    s0 = s0_ref[0, sl][:, None]
    s1 = s1_ref[0, sl][:, None]
    p0 = p0_ref[0, sl][:, None]
    p1 = p1_ref[0, sl][:, None]
    cols = lax.broadcasted_iota(jnp.int32, (n, S), 1)
    w = jnp.where(cols == s0, p0, 0.0) + jnp.where(cols == s1, p1, 0.0)
    w_ref[...] = w
    m_ref[...] = w != 0.0


def _dw_call(s0, s1, p0, p1):
    blk = 128
    grid = T // blk
    return pl.pallas_call(
        _dw_body,
        grid=(grid,),
        in_specs=[
            pl.BlockSpec((1, T), lambda i: (0, 0)),
            pl.BlockSpec((1, T), lambda i: (0, 0)),
            pl.BlockSpec((1, T), lambda i: (0, 0)),
            pl.BlockSpec((1, T), lambda i: (0, 0)),
        ],
        out_specs=[
            pl.BlockSpec((blk, S), lambda i: (i, 0)),
            pl.BlockSpec((blk, S), lambda i: (i, 0)),
        ],
        out_shape=[
            jax.ShapeDtypeStruct((T, S), jnp.float32),
            jax.ShapeDtypeStruct((T, S), jnp.bool_),
        ],
    )(s0, s1, p0, p1)


def _sc_body(s_hbm, x_hbm, z_hbm, out_hbm, slist_v, inv_v, rows_v):
    cid = lax.axis_index("c")
    sid = lax.axis_index("s")
    wid = sid * 2 + cid
    base = wid * SPW

    pltpu.sync_copy(s_hbm, slist_v)

    def init_inv(i, _):
        inv_v[pl.ds(i * 16, 16)] = jnp.full((16,), -1, jnp.int32)
        return 0
    lax.fori_loop(0, SPW // 16, init_inv, 0)

    def scan_assignments(i, _):
        s = slist_v[pl.ds(i * 16, 16)]
        a = i * 16 + lax.iota(jnp.int32, 16)
        t = lax.rem(a, T)
        m = (s >= base) & (s < base + SPW)
        idx = jnp.where(m, s - base, 0)
        plsc.store_scatter(inv_v, [idx], t, mask=m)
        return 0
    lax.fori_loop(0, (K * T) // 16, scan_assignments, 0)

    for j in range(SPW // CHUNK):
        inv_chunk = inv_v[pl.ds(j * CHUNK, CHUNK)]
        idx = jnp.maximum(inv_chunk, 0)
        pltpu.sync_copy(x_hbm.at[idx], rows_v)

        for r in range(CHUNK):
            tok = inv_chunk[r]
            @pl.when(tok < 0)
            def _():
                pltpu.sync_copy(z_hbm.at[0], rows_v.at[r])

        pltpu.sync_copy(rows_v, out_hbm.at[pl.ds(base + j * CHUNK, CHUNK)])


@functools.cache
def _get_sc_call():
    return pl.kernel(
        _sc_body,
        out_type=jax.ShapeDtypeStruct((S, D), jnp.float32),
        mesh=plsc.VectorSubcoreMesh(core_axis_name="c", subcore_axis_name="s"),
        compiler_params=pltpu.CompilerParams(needs_layout_passes=False),
        scratch_types=[
            pltpu.VMEM((K * T,), jnp.int32),
            pltpu.VMEM((SPW,), jnp.int32),
            pltpu.VMEM((CHUNK, D), jnp.float32),
        ],
    )


def kernel(x, W_g):
    xf = x.reshape(T, D)
    e0, e1, p0, p1 = _m1_call(xf, W_g)
    s0, s1 = _m2_call(e0, e1, p0, p1)
    w2d, m2d = _dw_call(s0, s1, p0, p1)
    s_flat = jnp.concatenate([s0, s1], axis=0).reshape(K * T)
    zrow = jnp.zeros((8, D), jnp.float32)
    eb = _get_sc_call()(s_flat, xf, zrow)
    return (
        w2d.reshape(T, E, CAP),
        m2d.reshape(T, E, CAP),
        eb.reshape(E, CAP, D),
    )
